# baseline (speedup 1.0000x reference)
import jax
import jax.numpy as jnp
import numpy as np
from jax.experimental import pallas as pl

_GROUPS = 8


def _router_body(x_ref, wbd_ref, s_ref, o_ref):
    l = jnp.dot(x_ref[...], wbd_ref[...], preferred_element_type=jnp.float32)
    m = jnp.max(l, axis=-1, keepdims=True)
    e = jnp.exp(l - m)
    s = jnp.dot(e, s_ref[...], preferred_element_type=jnp.float32)
    o_ref[...] = e / s


def kernel(x, expert_embeddings):
    B, S, H, D = x.shape
    E = expert_embeddings.shape[1]
    G = _GROUPS
    R = B * S * H // G
    K = G * D
    N = G * E
    x2 = x.reshape(R, K)

    eye = jnp.eye(G, dtype=x.dtype)
    wbd = jnp.einsum("gh,de->gdhe", eye, expert_embeddings).reshape(K, N)
    smat = jnp.asarray(
        np.kron(np.eye(G, dtype=np.float32), np.ones((E, E), dtype=np.float32))
    )

    BLK = 1024
    out = pl.pallas_call(
        _router_body,
        grid=(R // BLK,),
        in_specs=[
            pl.BlockSpec((BLK, K), lambda i: (i, 0)),
            pl.BlockSpec((K, N), lambda i: (0, 0)),
            pl.BlockSpec((N, N), lambda i: (0, 0)),
        ],
        out_specs=pl.BlockSpec((BLK, N), lambda i: (i, 0)),
        out_shape=jax.ShapeDtypeStruct((R, N), jnp.float32),
    )(x2, wbd, smat)
    return out.reshape(B, S, H, E)


# speedup vs baseline: 1.7682x; 1.7682x over previous
import jax
import jax.numpy as jnp
from jax.experimental import pallas as pl


def _router_body(x_ref, w_ref, o_ref):
    logits = jnp.dot(x_ref[...], w_ref[...], preferred_element_type=jnp.float32)
    m = jnp.max(logits, axis=-1, keepdims=True)
    e = jnp.exp(logits - m)
    o_ref[...] = e / jnp.sum(e, axis=-1, keepdims=True)


def kernel(x, expert_embeddings):
    B, S, H, D = x.shape
    E = expert_embeddings.shape[1]
    R = B * S * H
    x2 = x.reshape(R, D)
    BLK = 8192
    out = pl.pallas_call(
        _router_body,
        grid=(R // BLK,),
        in_specs=[
            pl.BlockSpec((BLK, D), lambda i: (i, 0)),
            pl.BlockSpec((D, E), lambda i: (0, 0)),
        ],
        out_specs=pl.BlockSpec((BLK, E), lambda i: (i, 0)),
        out_shape=jax.ShapeDtypeStruct((R, E), jnp.float32),
    )(x2, expert_embeddings)
    return out.reshape(B, S, H, E)


# speedup vs baseline: 1.8257x; 1.0325x over previous
import jax
import jax.numpy as jnp
from jax.experimental import pallas as pl


def _router_body(x_ref, w_ref, o_ref):
    logits = jnp.dot(x_ref[...], w_ref[...], preferred_element_type=jnp.float32)
    m = jnp.max(logits, axis=-1, keepdims=True)
    e = jnp.exp(logits - m)
    o_ref[...] = e / jnp.sum(e, axis=-1, keepdims=True)


def kernel(x, expert_embeddings):
    B, S, H, D = x.shape
    E = expert_embeddings.shape[1]
    R = B * S * H
    x2 = x.reshape(R, D)
    BLK = 16384
    out = pl.pallas_call(
        _router_body,
        grid=(R // BLK,),
        in_specs=[
            pl.BlockSpec((BLK, D), lambda i: (i, 0)),
            pl.BlockSpec((D, E), lambda i: (0, 0)),
        ],
        out_specs=pl.BlockSpec((BLK, E), lambda i: (i, 0)),
        out_shape=jax.ShapeDtypeStruct((R, E), jnp.float32),
    )(x2, expert_embeddings)
    return out.reshape(B, S, H, E)
